# Initial kernel scaffold; baseline (speedup 1.0000x reference)
import jax
import jax.numpy as jnp
from jax.experimental import pallas as pl


def kernel(x, edge_index, W_gcn, b_gcn, W_lin, b_lin, gamma, beta):
    raise NotImplementedError("write your pallas kernel here")


# speedup vs baseline: 3.8427x; 3.8427x over previous
import functools

import jax
import jax.numpy as jnp
from jax import lax
from jax.experimental import pallas as pl
from jax.experimental.pallas import tpu as pltpu
from jax.experimental.pallas import tpu_sc as plsc

N_NODES = 10000
N_EDGES = 320000
D = 128
DH = D // 2
BN_EPS = 1e-5

NC, NS = 2, 16
CHUNK = 128
E_PAD = 327680
EDGE_ROWS = E_PAD // CHUNK
TILE_ROWS = EDGE_ROWS // NS
N_PAD = 10240
ROWS_PT = N_PAD // NS
STAGE = 128

_sc_mesh = plsc.VectorSubcoreMesh(core_axis_name="c", subcore_axis_name="s")
_sc_params = pltpu.CompilerParams(use_tc_tiling_on_sc=False)


@functools.partial(
    pl.kernel,
    out_type=jax.ShapeDtypeStruct((NC, N_PAD, 16), jnp.float32),
    mesh=_sc_mesh,
    compiler_params=_sc_params,
    scratch_types=[
        pltpu.VMEM((TILE_ROWS, CHUNK), jnp.int32),
        pltpu.VMEM((CHUNK, 16), jnp.float32),
        pltpu.VMEM((ROWS_PT, 16), jnp.float32),
        pltpu.VMEM_SHARED((N_PAD, 16), jnp.float32),
    ],
)
def _degree_kernel(edge_hbm, out_hbm, idx_v, ones_v, stage_v, deg_sh):
    c = lax.axis_index("c")
    s = lax.axis_index("s")

    def fill(r, _):
        ones_v[r, :] = jnp.ones((16,), jnp.float32)
        return _

    lax.fori_loop(0, CHUNK, fill, 0)

    def zrow(r, _):
        stage_v[r, :] = jnp.zeros((16,), jnp.float32)
        return _

    lax.fori_loop(0, ROWS_PT, zrow, 0)
    pltpu.sync_copy(stage_v, deg_sh.at[pl.ds(s * ROWS_PT, ROWS_PT)])
    plsc.subcore_barrier()

    pltpu.sync_copy(edge_hbm.at[c, pl.ds(s * TILE_ROWS, TILE_ROWS)], idx_v)

    def body(j, _):
        pltpu.sync_copy(ones_v, deg_sh.at[idx_v.at[j]], add=True)
        return _

    lax.fori_loop(0, TILE_ROWS, body, 0)
    plsc.subcore_barrier()
    pltpu.sync_copy(deg_sh.at[pl.ds(s * ROWS_PT, ROWS_PT)], stage_v)
    pltpu.sync_copy(stage_v, out_hbm.at[c, pl.ds(s * ROWS_PT, ROWS_PT)])


@functools.partial(
    pl.kernel,
    out_type=jax.ShapeDtypeStruct((NC, N_PAD, DH), jnp.float32),
    mesh=_sc_mesh,
    compiler_params=_sc_params,
    scratch_types=[
        pltpu.VMEM((TILE_ROWS, CHUNK), jnp.int32),
        pltpu.VMEM((TILE_ROWS, CHUNK), jnp.int32),
        pltpu.VMEM((CHUNK, DH), jnp.float32),
        pltpu.VMEM((STAGE, DH), jnp.float32),
        pltpu.VMEM_SHARED((N_PAD, DH), jnp.float32),
        pltpu.SemaphoreType.DMA,
    ],
)
def _segsum_kernel(u_hbm, src_hbm, dst_hbm, out_hbm,
                   src_v, dst_v, rows_v, stage_v, agg_sh, sem):
    c = lax.axis_index("c")
    s = lax.axis_index("s")

    def zrow(r, _):
        for k in range(DH // 16):
            stage_v[r, pl.ds(k * 16, 16)] = jnp.zeros((16,), jnp.float32)
        return _

    lax.fori_loop(0, STAGE, zrow, 0)
    for i in range(ROWS_PT // STAGE):
        pltpu.sync_copy(stage_v, agg_sh.at[pl.ds(s * ROWS_PT + i * STAGE, STAGE)])
    plsc.subcore_barrier()

    pltpu.sync_copy(src_hbm.at[pl.ds(s * TILE_ROWS, TILE_ROWS)], src_v)
    pltpu.sync_copy(dst_hbm.at[pl.ds(s * TILE_ROWS, TILE_ROWS)], dst_v)

    def body(j, _):
        pltpu.async_copy(u_hbm.at[c].at[src_v.at[j]], rows_v, sem).wait()
        pltpu.sync_copy(rows_v, agg_sh.at[dst_v.at[j]], add=True)
        return _

    lax.fori_loop(0, TILE_ROWS, body, 0)
    plsc.subcore_barrier()
    for i in range(ROWS_PT // STAGE):
        pltpu.sync_copy(agg_sh.at[pl.ds(s * ROWS_PT + i * STAGE, STAGE)], stage_v)
        pltpu.sync_copy(stage_v, out_hbm.at[c, pl.ds(s * ROWS_PT + i * STAGE, STAGE)])


def _norm_col(deg16):
    d = deg16[:N_NODES, 0:1]
    return jnp.where(d > 0, lax.rsqrt(jnp.maximum(d, 1e-12)), 0.0)


def _split_u(u_ref, un):
    u_ref[0, :N_NODES, :] = un[:, :DH]
    u_ref[1, :N_NODES, :] = un[:, DH:]
    u_ref[0, N_NODES:, :] = jnp.zeros((N_PAD - N_NODES, DH), jnp.float32)
    u_ref[1, N_NODES:, :] = jnp.zeros((N_PAD - N_NODES, DH), jnp.float32)


def _scale_body(deg_ref, x_ref, u_ref):
    un = x_ref[...] * _norm_col(deg_ref[0])
    _split_u(u_ref, un)


_scale_call = pl.pallas_call(
    _scale_body,
    out_shape=jax.ShapeDtypeStruct((NC, N_PAD, DH), jnp.float32),
)


def _post_body(with_u, aggp_ref, h_ref, deg_ref, wg_ref, bg_ref, wl_ref,
               bl_ref, gm_ref, bt_ref, *outs):
    agg = jnp.concatenate(
        [aggp_ref[0][:N_NODES], aggp_ref[1][:N_NODES]], axis=1)
    h = h_ref[...]
    norm_in = _norm_col(deg_ref[1])
    gcn = jnp.dot(agg, wg_ref[...], preferred_element_type=jnp.float32)
    gcn = gcn * norm_in + bg_ref[...]
    lin = jnp.dot(h, wl_ref[...], preferred_element_type=jnp.float32) + bl_ref[...]
    out = gcn + h + lin
    mean = jnp.mean(out, axis=0, keepdims=True)
    cen = out - mean
    var = jnp.mean(cen * cen, axis=0, keepdims=True)
    hn = cen * lax.rsqrt(var + BN_EPS) * gm_ref[...] + bt_ref[...]
    hn = jnp.maximum(hn, 0.0)
    outs[0][...] = hn
    if with_u:
        _split_u(outs[1], hn * _norm_col(deg_ref[0]))


_post_first = pl.pallas_call(
    functools.partial(_post_body, True),
    out_shape=[jax.ShapeDtypeStruct((N_NODES, D), jnp.float32),
               jax.ShapeDtypeStruct((NC, N_PAD, DH), jnp.float32)],
)
_post_last = pl.pallas_call(
    functools.partial(_post_body, False),
    out_shape=[jax.ShapeDtypeStruct((N_NODES, D), jnp.float32)],
)


def kernel(x, edge_index, W_gcn, b_gcn, W_lin, b_lin, gamma, beta):
    pad = jnp.full((2, E_PAD - N_EDGES), N_NODES, dtype=edge_index.dtype)
    edge_r = jnp.concatenate([edge_index, pad], axis=1).reshape(
        2, EDGE_ROWS, CHUNK)
    src_r = edge_r[0]
    dst_r = edge_r[1]
    deg16 = _degree_kernel(edge_r)
    u = _scale_call(deg16, x)
    h = x
    for l in range(2):
        aggp = _segsum_kernel(u, src_r, dst_r)
        args = (aggp, h, deg16, W_gcn[l], b_gcn[l].reshape(1, D),
                W_lin[l], b_lin[l].reshape(1, D), gamma[l].reshape(1, D),
                beta[l].reshape(1, D))
        if l == 0:
            h, u = _post_first(*args)
        else:
            (h,) = _post_last(*args)
    return h


# speedup vs baseline: 5.0211x; 1.3067x over previous
import functools

import jax
import jax.numpy as jnp
from jax import lax
from jax.experimental import pallas as pl
from jax.experimental.pallas import tpu as pltpu
from jax.experimental.pallas import tpu_sc as plsc

N_NODES = 10000
N_EDGES = 320000
D = 128
DH = D // 2
BN_EPS = 1e-5

NC, NS = 2, 16
CHUNK = 128
E_PAD = 327680
EDGE_ROWS = E_PAD // CHUNK
TILE_ROWS = EDGE_ROWS // NS
N_PAD = 10240
ROWS_PT = N_PAD // NS
STAGE = 128
NBUF = 4

_sc_mesh = plsc.VectorSubcoreMesh(core_axis_name="c", subcore_axis_name="s")
_sc_params = pltpu.CompilerParams(use_tc_tiling_on_sc=False)


@functools.partial(
    pl.kernel,
    out_type=jax.ShapeDtypeStruct((NC, N_PAD, 16), jnp.float32),
    mesh=_sc_mesh,
    compiler_params=_sc_params,
    scratch_types=[
        pltpu.VMEM((TILE_ROWS, CHUNK), jnp.int32),
        pltpu.VMEM((CHUNK, 16), jnp.float32),
        pltpu.VMEM((ROWS_PT, 16), jnp.float32),
        pltpu.VMEM_SHARED((N_PAD, 16), jnp.float32),
    ],
)
def _degree_kernel(edge_hbm, out_hbm, idx_v, ones_v, stage_v, deg_sh):
    c = lax.axis_index("c")
    s = lax.axis_index("s")

    def fill(r, _):
        ones_v[r, :] = jnp.ones((16,), jnp.float32)
        return _

    lax.fori_loop(0, CHUNK, fill, 0)

    def zrow(r, _):
        stage_v[r, :] = jnp.zeros((16,), jnp.float32)
        return _

    lax.fori_loop(0, ROWS_PT, zrow, 0)
    pltpu.sync_copy(stage_v, deg_sh.at[pl.ds(s * ROWS_PT, ROWS_PT)])
    plsc.subcore_barrier()

    pltpu.sync_copy(edge_hbm.at[c, pl.ds(s * TILE_ROWS, TILE_ROWS)], idx_v)

    def body(j, _):
        pltpu.sync_copy(ones_v, deg_sh.at[idx_v.at[j]], add=True)
        return _

    lax.fori_loop(0, TILE_ROWS, body, 0)
    plsc.subcore_barrier()
    pltpu.sync_copy(deg_sh.at[pl.ds(s * ROWS_PT, ROWS_PT)], stage_v)
    pltpu.sync_copy(stage_v, out_hbm.at[c, pl.ds(s * ROWS_PT, ROWS_PT)])


@functools.partial(
    pl.kernel,
    out_type=jax.ShapeDtypeStruct((NC, N_PAD, DH), jnp.float32),
    mesh=_sc_mesh,
    compiler_params=_sc_params,
    scratch_types=[
        pltpu.VMEM((TILE_ROWS, CHUNK), jnp.int32),
        pltpu.VMEM((TILE_ROWS, CHUNK), jnp.int32),
        [pltpu.VMEM((CHUNK, DH), jnp.float32)] * NBUF,
        pltpu.VMEM((STAGE, DH), jnp.float32),
        pltpu.VMEM_SHARED((N_PAD, DH), jnp.float32),
        [pltpu.SemaphoreType.DMA] * NBUF,
    ],
)
def _segsum_kernel(u_hbm, src_hbm, dst_hbm, out_hbm,
                   src_v, dst_v, rows_bufs, stage_v, agg_sh, sems):
    c = lax.axis_index("c")
    s = lax.axis_index("s")

    def zrow(r, _):
        for k in range(DH // 16):
            stage_v[r, pl.ds(k * 16, 16)] = jnp.zeros((16,), jnp.float32)
        return _

    lax.fori_loop(0, STAGE, zrow, 0)
    for i in range(ROWS_PT // STAGE):
        pltpu.sync_copy(stage_v, agg_sh.at[pl.ds(s * ROWS_PT + i * STAGE, STAGE)])
    plsc.subcore_barrier()

    pltpu.sync_copy(src_hbm.at[pl.ds(s * TILE_ROWS, TILE_ROWS)], src_v)
    pltpu.sync_copy(dst_hbm.at[pl.ds(s * TILE_ROWS, TILE_ROWS)], dst_v)

    for b in range(NBUF):
        pltpu.async_copy(u_hbm.at[c].at[src_v.at[b]], rows_bufs[b], sems[b])

    def body(i, _):
        for b in range(NBUF):
            j = i * NBUF + b
            pltpu.make_async_copy(
                u_hbm.at[c].at[src_v.at[j]], rows_bufs[b], sems[b]).wait()
            pltpu.sync_copy(rows_bufs[b], agg_sh.at[dst_v.at[j]], add=True)

            @pl.when(j + NBUF < TILE_ROWS)
            def _issue():
                pltpu.async_copy(
                    u_hbm.at[c].at[src_v.at[j + NBUF]], rows_bufs[b], sems[b])
        return _

    lax.fori_loop(0, TILE_ROWS // NBUF, body, 0)
    plsc.subcore_barrier()
    for i in range(ROWS_PT // STAGE):
        pltpu.sync_copy(agg_sh.at[pl.ds(s * ROWS_PT + i * STAGE, STAGE)], stage_v)
        pltpu.sync_copy(stage_v, out_hbm.at[c, pl.ds(s * ROWS_PT + i * STAGE, STAGE)])


def _norm_col(deg16):
    d = deg16[:N_NODES, 0:1]
    return jnp.where(d > 0, lax.rsqrt(jnp.maximum(d, 1e-12)), 0.0)


def _split_u(u_ref, un):
    u_ref[0, :N_NODES, :] = un[:, :DH]
    u_ref[1, :N_NODES, :] = un[:, DH:]
    u_ref[0, N_NODES:, :] = jnp.zeros((N_PAD - N_NODES, DH), jnp.float32)
    u_ref[1, N_NODES:, :] = jnp.zeros((N_PAD - N_NODES, DH), jnp.float32)


def _scale_body(deg_ref, x_ref, u_ref):
    un = x_ref[...] * _norm_col(deg_ref[0])
    _split_u(u_ref, un)


_scale_call = pl.pallas_call(
    _scale_body,
    out_shape=jax.ShapeDtypeStruct((NC, N_PAD, DH), jnp.float32),
)


def _post_body(with_u, aggp_ref, h_ref, deg_ref, wg_ref, bg_ref, wl_ref,
               bl_ref, gm_ref, bt_ref, *outs):
    agg = jnp.concatenate(
        [aggp_ref[0][:N_NODES], aggp_ref[1][:N_NODES]], axis=1)
    h = h_ref[...]
    norm_in = _norm_col(deg_ref[1])
    gcn = jnp.dot(agg, wg_ref[...], preferred_element_type=jnp.float32)
    gcn = gcn * norm_in + bg_ref[...]
    lin = jnp.dot(h, wl_ref[...], preferred_element_type=jnp.float32) + bl_ref[...]
    out = gcn + h + lin
    mean = jnp.mean(out, axis=0, keepdims=True)
    cen = out - mean
    var = jnp.mean(cen * cen, axis=0, keepdims=True)
    hn = cen * lax.rsqrt(var + BN_EPS) * gm_ref[...] + bt_ref[...]
    hn = jnp.maximum(hn, 0.0)
    outs[0][...] = hn
    if with_u:
        _split_u(outs[1], hn * _norm_col(deg_ref[0]))


_post_first = pl.pallas_call(
    functools.partial(_post_body, True),
    out_shape=[jax.ShapeDtypeStruct((N_NODES, D), jnp.float32),
               jax.ShapeDtypeStruct((NC, N_PAD, DH), jnp.float32)],
)
_post_last = pl.pallas_call(
    functools.partial(_post_body, False),
    out_shape=[jax.ShapeDtypeStruct((N_NODES, D), jnp.float32)],
)


def kernel(x, edge_index, W_gcn, b_gcn, W_lin, b_lin, gamma, beta):
    pad = jnp.full((2, E_PAD - N_EDGES), N_NODES, dtype=edge_index.dtype)
    edge_r = jnp.concatenate([edge_index, pad], axis=1).reshape(
        2, EDGE_ROWS, CHUNK)
    src_r = edge_r[0]
    dst_r = edge_r[1]
    deg16 = _degree_kernel(edge_r)
    u = _scale_call(deg16, x)
    h = x
    for l in range(2):
        aggp = _segsum_kernel(u, src_r, dst_r)
        args = (aggp, h, deg16, W_gcn[l], b_gcn[l].reshape(1, D),
                W_lin[l], b_lin[l].reshape(1, D), gamma[l].reshape(1, D),
                beta[l].reshape(1, D))
        if l == 0:
            h, u = _post_first(*args)
        else:
            (h,) = _post_last(*args)
    return h
